# baseline (speedup 1.0000x reference)
import functools

import jax
import jax.numpy as jnp
from jax import lax
from jax.experimental import pallas as pl
from jax.experimental.pallas import tpu as pltpu
from jax.experimental.pallas import tpu_sc as plsc

_LEVELS = 1024
_OUT = 2048
_N = 224 * 224
_NW = 32
_PER_W = _N // _NW
_VPW = _PER_W // 16
_NLANES = 16
_TAB = _NLANES * _LEVELS
_RNE = 8388608.0

_mesh = plsc.VectorSubcoreMesh(core_axis_name="c", subcore_axis_name="s")


@functools.partial(
    pl.kernel,
    mesh=_mesh,
    out_type=jax.ShapeDtypeStruct((_NW, _TAB), jnp.float32),
    compiler_params=pltpu.CompilerParams(needs_layout_passes=False),
    scratch_types=[
        pltpu.VMEM((_PER_W,), jnp.float32),
        pltpu.VMEM((_TAB,), jnp.float32),
        pltpu.SemaphoreType.DMA,
    ],
)
def _sc_hist(x_hbm, out_hbm, x_v, tab_v, sem):
    wid = lax.axis_index("s") * 2 + lax.axis_index("c")
    base = wid * _PER_W
    cp = pltpu.async_copy(x_hbm.at[pl.ds(base, _PER_W)], x_v, sem)

    zeros16 = jnp.zeros((16,), jnp.float32)

    def _zero(j, c):
        for t in range(_NLANES):
            tab_v[pl.ds(t * _LEVELS + j * 16, 16)] = zeros16
        return c

    lax.fori_loop(0, _LEVELS // 16, _zero, 0)
    cp.wait()

    lane_base = lax.iota(jnp.int32, 16) * _LEVELS
    ones16 = jnp.ones((16,), jnp.float32)

    def _hist(i, c):
        for u in range(2):
            xv = x_v[pl.ds((2 * i + u) * 16, 16)]
            v = xv * float(_LEVELS - 1)
            r = (v + _RNE) - _RNE
            idx = jnp.clip(r.astype(jnp.int32), 0, _LEVELS - 1)
            plsc.addupdate_scatter(tab_v, [lane_base + idx], ones16)
        return c

    lax.fori_loop(0, _VPW // 2, _hist, 0)

    pltpu.sync_copy(tab_v, out_hbm.at[wid])


def _mv_body(cp_ref, w_ref, o_ref):
    counts = jnp.sum(cp_ref[...], axis=0, keepdims=True)
    o_ref[...] = jnp.dot(counts, w_ref[...], preferred_element_type=jnp.float32)


def kernel(x, level_weight):
    tables = _sc_hist(x)
    tables = tables.reshape(_NW * _NLANES, _LEVELS)
    out = pl.pallas_call(
        _mv_body,
        out_shape=jax.ShapeDtypeStruct((1, _OUT), jnp.float32),
    )(tables, level_weight)
    return out.reshape(_OUT)


# speedup vs baseline: 1.1166x; 1.1166x over previous
import functools

import jax
import jax.numpy as jnp
from jax import lax
from jax.experimental import pallas as pl
from jax.experimental.pallas import tpu as pltpu
from jax.experimental.pallas import tpu_sc as plsc

_LEVELS = 1024
_OUT = 2048
_N = 224 * 224
_NW = 32
_PER_W = _N // _NW
_VPW = _PER_W // 16
_NLANES = 16
_TAB = _NLANES * _LEVELS
_RNE = 8388608.0

_mesh = plsc.VectorSubcoreMesh(core_axis_name="c", subcore_axis_name="s")


@functools.partial(
    pl.kernel,
    mesh=_mesh,
    out_type=jax.ShapeDtypeStruct((_NW, _LEVELS), jnp.float32),
    compiler_params=pltpu.CompilerParams(needs_layout_passes=False),
    scratch_types=[
        pltpu.VMEM((_PER_W,), jnp.float32),
        pltpu.VMEM((_TAB,), jnp.float32),
        pltpu.VMEM((_LEVELS,), jnp.float32),
        pltpu.SemaphoreType.DMA,
    ],
)
def _sc_hist(x_hbm, out_hbm, x_v, tab_v, cnt_v, sem):
    wid = lax.axis_index("s") * 2 + lax.axis_index("c")
    base = wid * _PER_W
    cp = pltpu.async_copy(x_hbm.at[pl.ds(base, _PER_W)], x_v, sem)

    zeros16 = jnp.zeros((16,), jnp.float32)

    def _zero(j, c):
        for t in range(_NLANES):
            tab_v[pl.ds(t * _LEVELS + j * 16, 16)] = zeros16
        return c

    lax.fori_loop(0, _LEVELS // 16, _zero, 0)
    cp.wait()

    lane_base = lax.iota(jnp.int32, 16) * _LEVELS
    ones16 = jnp.ones((16,), jnp.float32)

    def _hist(i, c):
        for u in range(2):
            xv = x_v[pl.ds((2 * i + u) * 16, 16)]
            v = xv * float(_LEVELS - 1)
            r = (v + _RNE) - _RNE
            idx = jnp.clip(r.astype(jnp.int32), 0, _LEVELS - 1)
            plsc.addupdate_scatter(tab_v, [lane_base + idx], ones16)
        return c

    lax.fori_loop(0, _VPW // 2, _hist, 0)

    def _red(j, c):
        acc = tab_v[pl.ds(j * 16, 16)]
        for t in range(1, _NLANES):
            acc = acc + tab_v[pl.ds(t * _LEVELS + j * 16, 16)]
        cnt_v[pl.ds(j * 16, 16)] = acc
        return c

    lax.fori_loop(0, _LEVELS // 16, _red, 0)

    pltpu.sync_copy(cnt_v, out_hbm.at[wid])


def _mv_body(cp_ref, w_ref, o_ref):
    counts = jnp.sum(cp_ref[...], axis=0, keepdims=True)
    o_ref[...] = jnp.dot(counts, w_ref[...], preferred_element_type=jnp.float32)


def kernel(x, level_weight):
    counts_parts = _sc_hist(x)
    out = pl.pallas_call(
        _mv_body,
        out_shape=jax.ShapeDtypeStruct((1, _OUT), jnp.float32),
    )(counts_parts, level_weight)
    return out.reshape(_OUT)
